# baseline (speedup 1.0000x reference)
import functools

import jax
import jax.numpy as jnp
from jax import lax
from jax.experimental import pallas as pl
from jax.experimental.pallas import tpu as pltpu


def _emb_body(B, D, word_smem, table_hbm, out_vmem, sem):
    UNROLL = 16

    for i in range(B):
        idx = word_smem[i]
        pltpu.make_async_copy(
            table_hbm.at[pl.ds(idx, 1), :],
            out_vmem.at[pl.ds(i, 1), :],
            sem,
        ).start(priority=i % 2)
    pltpu.make_async_copy(table_hbm.at[pl.ds(0, B), :], out_vmem, sem).wait()


def kernel(word, table):
    (B,) = word.shape
    _, D = table.shape

    out = pl.pallas_call(
        functools.partial(_emb_body, B, D),
        in_specs=[
            pl.BlockSpec(memory_space=pltpu.SMEM),
            pl.BlockSpec(memory_space=pl.ANY),
        ],
        out_specs=pl.BlockSpec(memory_space=pltpu.VMEM),
        out_shape=jax.ShapeDtypeStruct((B, D), jnp.float32),
        scratch_shapes=[pltpu.SemaphoreType.DMA],
    )(word, table)
    return out.reshape(1, 1, -1)


# speedup vs baseline: 1.0219x; 1.0219x over previous
import functools

import jax
import jax.numpy as jnp
from jax import lax
from jax.experimental import pallas as pl
from jax.experimental.pallas import tpu as pltpu


def _emb_body(B, D, word_smem, table_hbm, out_hbm, rows_vmem, sem, wsem):
    UNROLL = 16

    for i in range(B):
        idx = word_smem[i]
        pltpu.make_async_copy(
            table_hbm.at[pl.ds(idx, 1), :],
            rows_vmem.at[pl.ds(i, 1), :],
            sem,
        ).start(priority=i % 2)
    H = B // 2
    pltpu.make_async_copy(
        table_hbm.at[pl.ds(0, H), :], rows_vmem.at[pl.ds(0, H), :], sem
    ).wait()
    pltpu.make_async_copy(
        rows_vmem.at[pl.ds(0, H), :], out_hbm.at[pl.ds(0, H), :], wsem
    ).start()
    pltpu.make_async_copy(
        table_hbm.at[pl.ds(0, H), :], rows_vmem.at[pl.ds(H, H), :], sem
    ).wait()
    pltpu.make_async_copy(
        rows_vmem.at[pl.ds(H, H), :], out_hbm.at[pl.ds(H, H), :], wsem
    ).start()
    pltpu.make_async_copy(rows_vmem, out_hbm, wsem).wait()


def kernel(word, table):
    (B,) = word.shape
    _, D = table.shape

    out = pl.pallas_call(
        functools.partial(_emb_body, B, D),
        in_specs=[
            pl.BlockSpec(memory_space=pltpu.SMEM),
            pl.BlockSpec(memory_space=pl.ANY),
        ],
        out_specs=pl.BlockSpec(memory_space=pl.ANY),
        out_shape=jax.ShapeDtypeStruct((B, D), jnp.float32),
        scratch_shapes=[
            pltpu.VMEM((B, D), jnp.float32),
            pltpu.SemaphoreType.DMA,
            pltpu.SemaphoreType.DMA,
        ],
    )(word, table)
    return out.reshape(1, 1, -1)
